# baseline (speedup 1.0000x reference)
import functools

import jax
import jax.numpy as jnp
from jax import lax
from jax.experimental import pallas as pl
from jax.experimental.pallas import tpu as pltpu
from jax.experimental.pallas import tpu_sc as plsc

RADIUS = 0.2
NBLK = 256
MBLK = 256
_INTERPRET = False


def _select_body(n_total, xyzt_ref, pts_ref,
                 gidx_ref, table_ref, xyzp_ref):
    b = pl.program_id(0)
    nb = pl.program_id(1)
    xyzt = xyzt_ref[0]
    n0 = nb * NBLK
    xn = jnp.transpose(xyzt_ref[0, :, pl.ds(n0, NBLK)], (1, 0))
    judge = jnp.float32(RADIUS * RADIUS)
    big = jnp.float32(1e10)
    dx = xyzt[0:1, :] - xn[:, 0:1]
    dy = xyzt[1:2, :] - xn[:, 1:2]
    dz = xyzt[2:3, :] - xn[:, 2:3]
    dist = (dx * dx + dy * dy) + dz * dz
    db = jnp.where((dist > 1e-10) & (dist < judge), dist, big)
    mx, my, mz = dx >= 0, dy >= 0, dz >= 0
    a1 = jnp.where(mx, db, big)
    a0 = jnp.where(mx, big, db)
    b00 = jnp.where(my, big, a0)
    b01 = jnp.where(my, a0, big)
    b10 = jnp.where(my, big, a1)
    b11 = jnp.where(my, a1, big)
    leaves = []
    for bb in (b00, b01, b10, b11):
        leaves.append(jnp.where(mz, big, bb))
        leaves.append(jnp.where(mz, bb, big))
    nlanes = 128
    nch = n_total // nlanes
    lane_iota = lax.broadcasted_iota(jnp.int32, (NBLK, nlanes), 1)
    nglob = nb * NBLK + lax.broadcasted_iota(jnp.int32, (NBLK, 1), 0)
    cols = []
    for lf in leaves:
        best = lf[:, 0:nlanes]
        colarg = jnp.zeros((NBLK, nlanes), jnp.int32)
        for c in range(1, nch):
            v = lf[:, c * nlanes:(c + 1) * nlanes]
            lt = v < best
            best = jnp.where(lt, v, best)
            colarg = jnp.where(lt, jnp.int32(c), colarg)
        mv = jnp.min(best, axis=1, keepdims=True)
        im = jnp.min(jnp.where(best == mv, colarg * nlanes + lane_iota,
                               jnp.int32(n_total)), axis=1, keepdims=True)
        cols.append(jnp.where(mv < judge, im, nglob))
    idx = jnp.concatenate(cols, axis=1)
    idxt = jnp.transpose(idx + b * n_total, (1, 0))
    gidx_ref[...] = jnp.transpose(
        jnp.reshape(idxt, (8, NBLK // 128, 128)), (1, 0, 2))
    zpad13 = jnp.zeros((NBLK, 13), jnp.float32)
    zpad48 = jnp.zeros((NBLK, 48), jnp.float32)
    xyzp = jnp.concatenate([xn, zpad13], axis=1)
    xyzp_ref[...] = xyzp
    table_ref[...] = jnp.concatenate([xyzp, pts_ref[...], zpad48], axis=1)


def _select(xyzt, pts_flat, b0, bh):
    n = xyzt.shape[2]
    rows = bh * n
    nb_per_b = n // NBLK
    grid = (bh, nb_per_b)
    return pl.pallas_call(
        functools.partial(_select_body, n),
        grid=grid,
        in_specs=[
            pl.BlockSpec((1, 3, n), lambda b, nb: (b0 + b, 0, 0)),
            pl.BlockSpec((NBLK, 64),
                         lambda b, nb: ((b0 + b) * nb_per_b + nb, 0)),
        ],
        out_specs=[
            pl.BlockSpec((NBLK // 128, 8, 128),
                         lambda b, nb: (b * nb_per_b + nb, 0, 0)),
            pl.BlockSpec((NBLK, 128), lambda b, nb: (b * nb_per_b + nb, 0)),
            pl.BlockSpec((NBLK, 16), lambda b, nb: (b * nb_per_b + nb, 0)),
        ],
        out_shape=[
            jax.ShapeDtypeStruct((rows // 128, 8, 128), jnp.int32),
            jax.ShapeDtypeStruct((rows, 128), jnp.float32),
            jax.ShapeDtypeStruct((rows, 16), jnp.float32),
        ],
        interpret=_INTERPRET,
    )(xyzt, pts_flat)


def _make_sc_gather(nslab, d):
    nw = 32
    nchunk_total = nslab * 8
    nchunk = nchunk_total // nw
    assert nchunk * 128 * d * 4 <= 500_000, "TileSpmem overflow"
    mesh = plsc.VectorSubcoreMesh(core_axis_name="c", subcore_axis_name="s")

    assert nchunk < 8 and 8 % nchunk == 0 or nchunk % 8 == 0
    idx_shape = ((nchunk // 8, 8, 128) if nchunk >= 8 else (nchunk, 128))

    @functools.partial(
        pl.kernel,
        mesh=mesh,
        out_type=jax.ShapeDtypeStruct((nchunk_total * 128, d), jnp.float32),
        scratch_types=[
            pltpu.VMEM(idx_shape, jnp.int32),
            pltpu.VMEM((nchunk * 128, d), jnp.float32),
            pltpu.SemaphoreType.DMA,
        ],
    )
    def gk(gidx_hbm, table_hbm, out_hbm, idx_v, rows_v, sem):
        wid = lax.axis_index("s") * 2 + lax.axis_index("c")
        if nchunk >= 8:
            pltpu.sync_copy(gidx_hbm.at[pl.ds(wid * (nchunk // 8),
                                              nchunk // 8)], idx_v)
        else:
            k0 = (wid * nchunk) // 8
            t0 = (wid * nchunk) % 8
            pltpu.sync_copy(gidx_hbm.at[k0, pl.ds(t0, nchunk)], idx_v)
        cps = [
            pltpu.async_copy(
                table_hbm.at[idx_v.at[c // 8, c % 8] if nchunk >= 8
                             else idx_v.at[c]],
                rows_v.at[pl.ds(c * 128, 128)], sem)
            for c in range(nchunk)
        ]
        for c in cps:
            c.wait()
        pltpu.sync_copy(rows_v,
                        out_hbm.at[pl.ds(wid * nchunk * 128, nchunk * 128)])

    return gk


def _sc_gather(gidx3, table):
    return _make_sc_gather(gidx3.shape[0], table.shape[1])(gidx3, table)


def _chain_core(g_ref, xyzp_ref, wa_ref, wb_ref, wc_ref,
                ba_ref, bb_ref, bc_ref):
    xyzp = xyzp_ref[...]
    c0 = (ba_ref[...]
          - jnp.dot(xyzp, wa_ref[0, 0:16, :],
                    preferred_element_type=jnp.float32)
          - jnp.dot(xyzp, wa_ref[1, 0:16, :],
                    preferred_element_type=jnp.float32))
    nsl = MBLK // 128
    x1 = []
    for w in range(4):
        taps = []
        for k in range(2):
            t = 2 * w + k
            taps.append(jnp.concatenate([g_ref[s, t] for s in range(nsl)],
                                        axis=0))
        s = (jnp.dot(taps[0], wa_ref[0], preferred_element_type=jnp.float32)
             + jnp.dot(taps[1], wa_ref[1],
                       preferred_element_type=jnp.float32))
        x1.append(s + c0)
    x2 = []
    for w in range(2):
        s = (jnp.dot(x1[2 * w], wb_ref[0], preferred_element_type=jnp.float32)
             + jnp.dot(x1[2 * w + 1], wb_ref[1],
                       preferred_element_type=jnp.float32))
        x2.append(s + bb_ref[...])
    return (jnp.dot(x2[0], wc_ref[0], preferred_element_type=jnp.float32)
            + jnp.dot(x2[1], wc_ref[1], preferred_element_type=jnp.float32)
            + bc_ref[...])


def _chain1_body(g_ref, xyzp_ref, wa_ref, wb_ref, wc_ref,
                 ba_ref, bb_ref, bc_ref, out_ref):
    x3 = _chain_core(g_ref, xyzp_ref, wa_ref, wb_ref, wc_ref,
                     ba_ref, bb_ref, bc_ref)
    zpad48 = jnp.zeros((x3.shape[0], 48), jnp.float32)
    out_ref[...] = jnp.concatenate([xyzp_ref[...], x3, zpad48], axis=1)


def _chain2_body(g_ref, xyzp_ref, wa_ref, wb_ref, wc_ref,
                 ba_ref, bb_ref, bc_ref, pts_ref, out_ref):
    x3 = _chain_core(g_ref, xyzp_ref, wa_ref, wb_ref, wc_ref,
                     ba_ref, bb_ref, bc_ref)
    out_ref[...] = jax.nn.relu(jnp.concatenate([x3, pts_ref[...]], axis=1))


def _run_chain(body, g4, xyzp, wa, wb, wc, ba, bb, bc,
               extra, extra_off, out_cols):
    rows = xyzp.shape[0]
    grid = (rows // MBLK,)
    nsl = MBLK // 128
    full = lambda i: (0, 0, 0)
    specs = [
        pl.BlockSpec((nsl, 8, 128, g4.shape[3]), lambda i: (i, 0, 0, 0)),
        pl.BlockSpec((MBLK, 16), lambda i: (i, 0)),
        pl.BlockSpec(wa.shape, full),
        pl.BlockSpec(wb.shape, full),
        pl.BlockSpec(wc.shape, full),
        pl.BlockSpec((1, 64), lambda i: (0, 0)),
        pl.BlockSpec((1, 64), lambda i: (0, 0)),
        pl.BlockSpec((1, 64), lambda i: (0, 0)),
    ]
    args = [g4, xyzp, wa, wb, wc, ba, bb, bc]
    if extra is not None:
        specs.append(pl.BlockSpec((MBLK, 64),
                                  lambda i: (extra_off + i, 0)))
        args.append(extra)
    return pl.pallas_call(
        body,
        grid=grid,
        in_specs=specs,
        out_specs=pl.BlockSpec((MBLK, out_cols), lambda i: (i, 0)),
        out_shape=jax.ShapeDtypeStruct((rows, out_cols), jnp.float32),
        interpret=_INTERPRET,
    )(*args)


def _prep_tap_weights(w):
    o, c, _ = w.shape
    out = jnp.zeros((2, 128, o), jnp.float32)
    wt = jnp.transpose(w, (2, 1, 0))
    out = out.at[:, 0:3, :].set(wt[:, 0:3, :])
    out = out.at[:, 16:16 + (c - 3), :].set(wt[:, 3:, :])
    return out


def kernel(xyz, points, w1a, b1a, w1b, b1b, w1c, b1c,
           w2a, b2a, w2b, b2b, w2c, b2c):
    B, T, N, _ = xyz.shape
    bt = B * T
    rows = bt * N
    xyzt = jnp.transpose(xyz.reshape(bt, N, 3), (0, 2, 1))
    pts_flat = points.reshape(rows, -1)

    wa1 = _prep_tap_weights(w1a)
    wb1 = jnp.transpose(w1b, (2, 1, 0))
    wc1 = jnp.transpose(w1c, (2, 1, 0))
    wa2 = _prep_tap_weights(w2a)
    wb2 = jnp.transpose(w2b, (2, 1, 0))
    wc2 = jnp.transpose(w2c, (2, 1, 0))

    ns = 2
    bh = bt // ns
    rh = bh * N
    merged_parts = []
    for s in range(ns):
        gidx3, table1, xyzp = _select(xyzt, pts_flat, s * bh, bh)
        g1 = _sc_gather(gidx3, table1).reshape(rh // 128, 8, 128, 128)
        table2 = _run_chain(_chain1_body, g1, xyzp, wa1, wb1, wc1,
                            b1a.reshape(1, -1), b1b.reshape(1, -1),
                            b1c.reshape(1, -1), None, 0, 128)
        g2 = _sc_gather(gidx3, table2).reshape(rh // 128, 8, 128, 128)
        merged_parts.append(
            _run_chain(_chain2_body, g2, xyzp, wa2, wb2, wc2,
                       b2a.reshape(1, -1), b2b.reshape(1, -1),
                       b2c.reshape(1, -1), pts_flat,
                       s * rh // MBLK, 128))
    merged = jnp.concatenate(merged_parts, axis=0)
    return (xyz, merged.reshape(B, T, N, 128))


# speedup vs baseline: 1.0696x; 1.0696x over previous
import functools

import jax
import jax.numpy as jnp
from jax import lax
from jax.experimental import pallas as pl
from jax.experimental.pallas import tpu as pltpu
from jax.experimental.pallas import tpu_sc as plsc

RADIUS = 0.2
NBLK = 256
MBLK = 512
_INTERPRET = False


def _select_body(n_total, xyzn_ref, xyzt_ref, pts_ref,
                 gidx_ref, table_ref, xyzp_ref):
    b = pl.program_id(0)
    nb = pl.program_id(1)
    xyzn = xyzn_ref[0]
    xyzt = xyzt_ref[0]
    judge = jnp.float32(RADIUS * RADIUS)
    big = jnp.float32(1e10)
    dx = xyzt[0:1, :] - xyzn[:, 0:1]
    dy = xyzt[1:2, :] - xyzn[:, 1:2]
    dz = xyzt[2:3, :] - xyzn[:, 2:3]
    dist = (dx * dx + dy * dy) + dz * dz
    db = jnp.where((dist > 1e-10) & (dist < judge), dist, big)
    mx, my, mz = dx >= 0, dy >= 0, dz >= 0
    a1 = jnp.where(mx, db, big)
    a0 = jnp.where(mx, big, db)
    b00 = jnp.where(my, big, a0)
    b01 = jnp.where(my, a0, big)
    b10 = jnp.where(my, big, a1)
    b11 = jnp.where(my, a1, big)
    leaves = []
    for bb in (b00, b01, b10, b11):
        leaves.append(jnp.where(mz, big, bb))
        leaves.append(jnp.where(mz, bb, big))
    nlanes = 128
    nch = n_total // nlanes
    lane_iota = lax.broadcasted_iota(jnp.int32, (NBLK, nlanes), 1)
    nglob = nb * NBLK + lax.broadcasted_iota(jnp.int32, (NBLK, 1), 0)
    cols = []
    for lf in leaves:
        best = lf[:, 0:nlanes]
        colarg = jnp.zeros((NBLK, nlanes), jnp.int32)
        for c in range(1, nch):
            v = lf[:, c * nlanes:(c + 1) * nlanes]
            lt = v < best
            best = jnp.where(lt, v, best)
            colarg = jnp.where(lt, jnp.int32(c), colarg)
        mv = jnp.min(best, axis=1, keepdims=True)
        im = jnp.min(jnp.where(best == mv, colarg * nlanes + lane_iota,
                               jnp.int32(n_total)), axis=1, keepdims=True)
        cols.append(jnp.where(mv < judge, im, nglob))
    idx = jnp.concatenate(cols, axis=1)
    idxt = jnp.transpose(idx + b * n_total, (1, 0))
    gidx_ref[...] = jnp.transpose(
        jnp.reshape(idxt, (8, NBLK // 128, 128)), (1, 0, 2))
    zpad13 = jnp.zeros((NBLK, 13), jnp.float32)
    zpad48 = jnp.zeros((NBLK, 48), jnp.float32)
    xyzp = jnp.concatenate([xyzn, zpad13], axis=1)
    xyzp_ref[...] = xyzp
    table_ref[...] = jnp.concatenate([xyzp, pts_ref[...], zpad48], axis=1)


def _select(xyz2, pts_flat, b0, bh):
    n = xyz2.shape[1]
    rows = bh * n
    xyzt = jnp.transpose(xyz2, (0, 2, 1))
    nb_per_b = n // NBLK
    grid = (bh, nb_per_b)
    return pl.pallas_call(
        functools.partial(_select_body, n),
        grid=grid,
        in_specs=[
            pl.BlockSpec((1, NBLK, 3), lambda b, nb: (b0 + b, nb, 0)),
            pl.BlockSpec((1, 3, n), lambda b, nb: (b0 + b, 0, 0)),
            pl.BlockSpec((NBLK, 64),
                         lambda b, nb: ((b0 + b) * nb_per_b + nb, 0)),
        ],
        out_specs=[
            pl.BlockSpec((NBLK // 128, 8, 128),
                         lambda b, nb: (b * nb_per_b + nb, 0, 0)),
            pl.BlockSpec((NBLK, 128), lambda b, nb: (b * nb_per_b + nb, 0)),
            pl.BlockSpec((NBLK, 16), lambda b, nb: (b * nb_per_b + nb, 0)),
        ],
        out_shape=[
            jax.ShapeDtypeStruct((rows // 128, 8, 128), jnp.int32),
            jax.ShapeDtypeStruct((rows, 128), jnp.float32),
            jax.ShapeDtypeStruct((rows, 16), jnp.float32),
        ],
        interpret=_INTERPRET,
    )(xyz2, xyzt, pts_flat)


def _make_sc_gather(nslab, d):
    nw = 32
    nchunk_total = nslab * 8
    nchunk = nchunk_total // nw
    assert nchunk * 128 * d * 4 <= 500_000, "TileSpmem overflow"
    mesh = plsc.VectorSubcoreMesh(core_axis_name="c", subcore_axis_name="s")

    assert nchunk < 8 and 8 % nchunk == 0 or nchunk % 8 == 0
    idx_shape = ((nchunk // 8, 8, 128) if nchunk >= 8 else (nchunk, 128))

    @functools.partial(
        pl.kernel,
        mesh=mesh,
        out_type=jax.ShapeDtypeStruct((nchunk_total * 128, d), jnp.float32),
        scratch_types=[
            pltpu.VMEM(idx_shape, jnp.int32),
            pltpu.VMEM((nchunk * 128, d), jnp.float32),
            pltpu.SemaphoreType.DMA,
        ],
    )
    def gk(gidx_hbm, table_hbm, out_hbm, idx_v, rows_v, sem):
        wid = lax.axis_index("s") * 2 + lax.axis_index("c")
        if nchunk >= 8:
            pltpu.sync_copy(gidx_hbm.at[pl.ds(wid * (nchunk // 8),
                                              nchunk // 8)], idx_v)
        else:
            k0 = (wid * nchunk) // 8
            t0 = (wid * nchunk) % 8
            pltpu.sync_copy(gidx_hbm.at[k0, pl.ds(t0, nchunk)], idx_v)
        cps = [
            pltpu.async_copy(
                table_hbm.at[idx_v.at[c // 8, c % 8] if nchunk >= 8
                             else idx_v.at[c]],
                rows_v.at[pl.ds(c * 128, 128)], sem)
            for c in range(nchunk)
        ]
        for c in cps:
            c.wait()
        pltpu.sync_copy(rows_v,
                        out_hbm.at[pl.ds(wid * nchunk * 128, nchunk * 128)])

    return gk


def _sc_gather(gidx3, table):
    return _make_sc_gather(gidx3.shape[0], table.shape[1])(gidx3, table)


def _chain_core(g_ref, xyzp_ref, wa_ref, wb_ref, wc_ref,
                ba_ref, bb_ref, bc_ref):
    xyzp = xyzp_ref[...]
    c0 = (ba_ref[...]
          - jnp.dot(xyzp, wa_ref[0, 0:16, :],
                    preferred_element_type=jnp.float32)
          - jnp.dot(xyzp, wa_ref[1, 0:16, :],
                    preferred_element_type=jnp.float32))
    nsl = MBLK // 128
    x1 = []
    for w in range(4):
        taps = []
        for k in range(2):
            t = 2 * w + k
            taps.append(jnp.concatenate([g_ref[s, t] for s in range(nsl)],
                                        axis=0))
        s = (jnp.dot(taps[0], wa_ref[0], preferred_element_type=jnp.float32)
             + jnp.dot(taps[1], wa_ref[1],
                       preferred_element_type=jnp.float32))
        x1.append(s + c0)
    x2 = []
    for w in range(2):
        s = (jnp.dot(x1[2 * w], wb_ref[0], preferred_element_type=jnp.float32)
             + jnp.dot(x1[2 * w + 1], wb_ref[1],
                       preferred_element_type=jnp.float32))
        x2.append(s + bb_ref[...])
    return (jnp.dot(x2[0], wc_ref[0], preferred_element_type=jnp.float32)
            + jnp.dot(x2[1], wc_ref[1], preferred_element_type=jnp.float32)
            + bc_ref[...])


def _chain1_body(g_ref, xyzp_ref, wa_ref, wb_ref, wc_ref,
                 ba_ref, bb_ref, bc_ref, out_ref):
    x3 = _chain_core(g_ref, xyzp_ref, wa_ref, wb_ref, wc_ref,
                     ba_ref, bb_ref, bc_ref)
    zpad48 = jnp.zeros((x3.shape[0], 48), jnp.float32)
    out_ref[...] = jnp.concatenate([xyzp_ref[...], x3, zpad48], axis=1)


def _chain2_body(g_ref, xyzp_ref, wa_ref, wb_ref, wc_ref,
                 ba_ref, bb_ref, bc_ref, pts_ref, out_ref):
    x3 = _chain_core(g_ref, xyzp_ref, wa_ref, wb_ref, wc_ref,
                     ba_ref, bb_ref, bc_ref)
    out_ref[...] = jax.nn.relu(jnp.concatenate([x3, pts_ref[...]], axis=1))


def _run_chain(body, g4, xyzp, wa, wb, wc, ba, bb, bc,
               extra, extra_off, out_cols):
    rows = xyzp.shape[0]
    grid = (rows // MBLK,)
    nsl = MBLK // 128
    full = lambda i: (0, 0, 0)
    specs = [
        pl.BlockSpec((nsl, 8, 128, g4.shape[3]), lambda i: (i, 0, 0, 0)),
        pl.BlockSpec((MBLK, 16), lambda i: (i, 0)),
        pl.BlockSpec(wa.shape, full),
        pl.BlockSpec(wb.shape, full),
        pl.BlockSpec(wc.shape, full),
        pl.BlockSpec((1, 64), lambda i: (0, 0)),
        pl.BlockSpec((1, 64), lambda i: (0, 0)),
        pl.BlockSpec((1, 64), lambda i: (0, 0)),
    ]
    args = [g4, xyzp, wa, wb, wc, ba, bb, bc]
    if extra is not None:
        specs.append(pl.BlockSpec((MBLK, 64),
                                  lambda i: (extra_off + i, 0)))
        args.append(extra)
    return pl.pallas_call(
        body,
        grid=grid,
        in_specs=specs,
        out_specs=pl.BlockSpec((MBLK, out_cols), lambda i: (i, 0)),
        out_shape=jax.ShapeDtypeStruct((rows, out_cols), jnp.float32),
        interpret=_INTERPRET,
    )(*args)


def _prep_tap_weights(w):
    o, c, _ = w.shape
    out = jnp.zeros((2, 128, o), jnp.float32)
    wt = jnp.transpose(w, (2, 1, 0))
    out = out.at[:, 0:3, :].set(wt[:, 0:3, :])
    out = out.at[:, 16:16 + (c - 3), :].set(wt[:, 3:, :])
    return out


def kernel(xyz, points, w1a, b1a, w1b, b1b, w1c, b1c,
           w2a, b2a, w2b, b2b, w2c, b2c):
    B, T, N, _ = xyz.shape
    bt = B * T
    rows = bt * N
    xyz2 = xyz.reshape(bt, N, 3)
    pts_flat = points.reshape(rows, -1)

    wa1 = _prep_tap_weights(w1a)
    wb1 = jnp.transpose(w1b, (2, 1, 0))
    wc1 = jnp.transpose(w1c, (2, 1, 0))
    wa2 = _prep_tap_weights(w2a)
    wb2 = jnp.transpose(w2b, (2, 1, 0))
    wc2 = jnp.transpose(w2c, (2, 1, 0))

    ns = 2
    bh = bt // ns
    rh = bh * N
    merged_parts = []
    for s in range(ns):
        gidx3, table1, xyzp = _select(xyz2, pts_flat, s * bh, bh)
        g1 = _sc_gather(gidx3, table1).reshape(rh // 128, 8, 128, 128)
        table2 = _run_chain(_chain1_body, g1, xyzp, wa1, wb1, wc1,
                            b1a.reshape(1, -1), b1b.reshape(1, -1),
                            b1c.reshape(1, -1), None, 0, 128)
        g2 = _sc_gather(gidx3, table2).reshape(rh // 128, 8, 128, 128)
        merged_parts.append(
            _run_chain(_chain2_body, g2, xyzp, wa2, wb2, wc2,
                       b2a.reshape(1, -1), b2b.reshape(1, -1),
                       b2c.reshape(1, -1), pts_flat,
                       s * rh // MBLK, 128))
    merged = jnp.concatenate(merged_parts, axis=0)
    return (xyz, merged.reshape(B, T, N, 128))


# speedup vs baseline: 1.1027x; 1.0310x over previous
import functools

import jax
import jax.numpy as jnp
from jax import lax
from jax.experimental import pallas as pl
from jax.experimental.pallas import tpu as pltpu
from jax.experimental.pallas import tpu_sc as plsc

RADIUS = 0.2
NBLK = 256
MBLK = 1024
_INTERPRET = False


def _select_body(n_total, xyzn_ref, xyzt_ref, pts_ref,
                 gidx_ref, table_ref, xyzp_ref):
    b = pl.program_id(0)
    nb = pl.program_id(1)
    xyzn = xyzn_ref[0]
    xyzt = xyzt_ref[0]
    judge = jnp.float32(RADIUS * RADIUS)
    big = jnp.float32(1e10)
    dx = xyzt[0:1, :] - xyzn[:, 0:1]
    dy = xyzt[1:2, :] - xyzn[:, 1:2]
    dz = xyzt[2:3, :] - xyzn[:, 2:3]
    dist = (dx * dx + dy * dy) + dz * dz
    db = jnp.where((dist > 1e-10) & (dist < judge), dist, big)
    mx, my, mz = dx >= 0, dy >= 0, dz >= 0
    a1 = jnp.where(mx, db, big)
    a0 = jnp.where(mx, big, db)
    b00 = jnp.where(my, big, a0)
    b01 = jnp.where(my, a0, big)
    b10 = jnp.where(my, big, a1)
    b11 = jnp.where(my, a1, big)
    leaves = []
    for bb in (b00, b01, b10, b11):
        leaves.append(jnp.where(mz, big, bb))
        leaves.append(jnp.where(mz, bb, big))
    nlanes = 128
    nch = n_total // nlanes
    lane_iota = lax.broadcasted_iota(jnp.int32, (NBLK, nlanes), 1)
    nglob = nb * NBLK + lax.broadcasted_iota(jnp.int32, (NBLK, 1), 0)
    cols = []
    for lf in leaves:
        best = lf[:, 0:nlanes]
        colarg = jnp.zeros((NBLK, nlanes), jnp.int32)
        for c in range(1, nch):
            v = lf[:, c * nlanes:(c + 1) * nlanes]
            lt = v < best
            best = jnp.where(lt, v, best)
            colarg = jnp.where(lt, jnp.int32(c), colarg)
        mv = jnp.min(best, axis=1, keepdims=True)
        im = jnp.min(jnp.where(best == mv, colarg * nlanes + lane_iota,
                               jnp.int32(n_total)), axis=1, keepdims=True)
        cols.append(jnp.where(mv < judge, im, nglob))
    idx = jnp.concatenate(cols, axis=1)
    idxt = jnp.transpose(idx + b * n_total, (1, 0))
    gidx_ref[...] = jnp.transpose(
        jnp.reshape(idxt, (8, NBLK // 128, 128)), (1, 0, 2))
    zpad13 = jnp.zeros((NBLK, 13), jnp.float32)
    zpad48 = jnp.zeros((NBLK, 48), jnp.float32)
    xyzp = jnp.concatenate([xyzn, zpad13], axis=1)
    xyzp_ref[...] = xyzp
    table_ref[...] = jnp.concatenate([xyzp, pts_ref[...], zpad48], axis=1)


def _select(xyz2, pts_flat, b0, bh):
    n = xyz2.shape[1]
    rows = bh * n
    xyzt = jnp.transpose(xyz2, (0, 2, 1))
    nb_per_b = n // NBLK
    grid = (bh, nb_per_b)
    return pl.pallas_call(
        functools.partial(_select_body, n),
        grid=grid,
        in_specs=[
            pl.BlockSpec((1, NBLK, 3), lambda b, nb: (b0 + b, nb, 0)),
            pl.BlockSpec((1, 3, n), lambda b, nb: (b0 + b, 0, 0)),
            pl.BlockSpec((NBLK, 64),
                         lambda b, nb: ((b0 + b) * nb_per_b + nb, 0)),
        ],
        out_specs=[
            pl.BlockSpec((NBLK // 128, 8, 128),
                         lambda b, nb: (b * nb_per_b + nb, 0, 0)),
            pl.BlockSpec((NBLK, 128), lambda b, nb: (b * nb_per_b + nb, 0)),
            pl.BlockSpec((NBLK, 16), lambda b, nb: (b * nb_per_b + nb, 0)),
        ],
        out_shape=[
            jax.ShapeDtypeStruct((rows // 128, 8, 128), jnp.int32),
            jax.ShapeDtypeStruct((rows, 128), jnp.float32),
            jax.ShapeDtypeStruct((rows, 16), jnp.float32),
        ],
        interpret=_INTERPRET,
    )(xyz2, xyzt, pts_flat)


def _make_sc_gather(nslab, d):
    nw = 32
    nchunk_total = nslab * 8
    nchunk = nchunk_total // nw
    assert nchunk * 128 * d * 4 <= 500_000, "TileSpmem overflow"
    mesh = plsc.VectorSubcoreMesh(core_axis_name="c", subcore_axis_name="s")

    assert nchunk < 8 and 8 % nchunk == 0 or nchunk % 8 == 0
    idx_shape = ((nchunk // 8, 8, 128) if nchunk >= 8 else (nchunk, 128))

    @functools.partial(
        pl.kernel,
        mesh=mesh,
        out_type=jax.ShapeDtypeStruct((nchunk_total * 128, d), jnp.float32),
        scratch_types=[
            pltpu.VMEM(idx_shape, jnp.int32),
            pltpu.VMEM((nchunk * 128, d), jnp.float32),
            pltpu.SemaphoreType.DMA,
        ],
    )
    def gk(gidx_hbm, table_hbm, out_hbm, idx_v, rows_v, sem):
        wid = lax.axis_index("s") * 2 + lax.axis_index("c")
        if nchunk >= 8:
            pltpu.sync_copy(gidx_hbm.at[pl.ds(wid * (nchunk // 8),
                                              nchunk // 8)], idx_v)
        else:
            k0 = (wid * nchunk) // 8
            t0 = (wid * nchunk) % 8
            pltpu.sync_copy(gidx_hbm.at[k0, pl.ds(t0, nchunk)], idx_v)
        cps = [
            pltpu.async_copy(
                table_hbm.at[idx_v.at[c // 8, c % 8] if nchunk >= 8
                             else idx_v.at[c]],
                rows_v.at[pl.ds(c * 128, 128)], sem)
            for c in range(nchunk)
        ]
        for c in cps:
            c.wait()
        pltpu.sync_copy(rows_v,
                        out_hbm.at[pl.ds(wid * nchunk * 128, nchunk * 128)])

    return gk


def _sc_gather(gidx3, table):
    return _make_sc_gather(gidx3.shape[0], table.shape[1])(gidx3, table)


def _chain_core(g_ref, xyzp_ref, wa_ref, wb_ref, wc_ref,
                ba_ref, bb_ref, bc_ref):
    xyzp = xyzp_ref[...]
    c0 = (ba_ref[...]
          - jnp.dot(xyzp, wa_ref[0, 0:16, :],
                    preferred_element_type=jnp.float32)
          - jnp.dot(xyzp, wa_ref[1, 0:16, :],
                    preferred_element_type=jnp.float32))
    nsl = MBLK // 128
    x1 = []
    for w in range(4):
        taps = []
        for k in range(2):
            t = 2 * w + k
            taps.append(jnp.concatenate([g_ref[s, t] for s in range(nsl)],
                                        axis=0))
        s = (jnp.dot(taps[0], wa_ref[0], preferred_element_type=jnp.float32)
             + jnp.dot(taps[1], wa_ref[1],
                       preferred_element_type=jnp.float32))
        x1.append(s + c0)
    x2 = []
    for w in range(2):
        s = (jnp.dot(x1[2 * w], wb_ref[0], preferred_element_type=jnp.float32)
             + jnp.dot(x1[2 * w + 1], wb_ref[1],
                       preferred_element_type=jnp.float32))
        x2.append(s + bb_ref[...])
    return (jnp.dot(x2[0], wc_ref[0], preferred_element_type=jnp.float32)
            + jnp.dot(x2[1], wc_ref[1], preferred_element_type=jnp.float32)
            + bc_ref[...])


def _chain1_body(g_ref, xyzp_ref, wa_ref, wb_ref, wc_ref,
                 ba_ref, bb_ref, bc_ref, out_ref):
    x3 = _chain_core(g_ref, xyzp_ref, wa_ref, wb_ref, wc_ref,
                     ba_ref, bb_ref, bc_ref)
    zpad48 = jnp.zeros((x3.shape[0], 48), jnp.float32)
    out_ref[...] = jnp.concatenate([xyzp_ref[...], x3, zpad48], axis=1)


def _chain2_body(g_ref, xyzp_ref, wa_ref, wb_ref, wc_ref,
                 ba_ref, bb_ref, bc_ref, pts_ref, out_ref):
    x3 = _chain_core(g_ref, xyzp_ref, wa_ref, wb_ref, wc_ref,
                     ba_ref, bb_ref, bc_ref)
    out_ref[...] = jax.nn.relu(jnp.concatenate([x3, pts_ref[...]], axis=1))


def _run_chain(body, g4, xyzp, wa, wb, wc, ba, bb, bc,
               extra, extra_off, out_cols):
    rows = xyzp.shape[0]
    grid = (rows // MBLK,)
    nsl = MBLK // 128
    full = lambda i: (0, 0, 0)
    specs = [
        pl.BlockSpec((nsl, 8, 128, g4.shape[3]), lambda i: (i, 0, 0, 0)),
        pl.BlockSpec((MBLK, 16), lambda i: (i, 0)),
        pl.BlockSpec(wa.shape, full),
        pl.BlockSpec(wb.shape, full),
        pl.BlockSpec(wc.shape, full),
        pl.BlockSpec((1, 64), lambda i: (0, 0)),
        pl.BlockSpec((1, 64), lambda i: (0, 0)),
        pl.BlockSpec((1, 64), lambda i: (0, 0)),
    ]
    args = [g4, xyzp, wa, wb, wc, ba, bb, bc]
    if extra is not None:
        specs.append(pl.BlockSpec((MBLK, 64),
                                  lambda i: (extra_off + i, 0)))
        args.append(extra)
    return pl.pallas_call(
        body,
        grid=grid,
        in_specs=specs,
        out_specs=pl.BlockSpec((MBLK, out_cols), lambda i: (i, 0)),
        out_shape=jax.ShapeDtypeStruct((rows, out_cols), jnp.float32),
        interpret=_INTERPRET,
    )(*args)


def _prep_tap_weights(w):
    o, c, _ = w.shape
    out = jnp.zeros((2, 128, o), jnp.float32)
    wt = jnp.transpose(w, (2, 1, 0))
    out = out.at[:, 0:3, :].set(wt[:, 0:3, :])
    out = out.at[:, 16:16 + (c - 3), :].set(wt[:, 3:, :])
    return out


def kernel(xyz, points, w1a, b1a, w1b, b1b, w1c, b1c,
           w2a, b2a, w2b, b2b, w2c, b2c):
    B, T, N, _ = xyz.shape
    bt = B * T
    rows = bt * N
    xyz2 = xyz.reshape(bt, N, 3)
    pts_flat = points.reshape(rows, -1)

    wa1 = _prep_tap_weights(w1a)
    wb1 = jnp.transpose(w1b, (2, 1, 0))
    wc1 = jnp.transpose(w1c, (2, 1, 0))
    wa2 = _prep_tap_weights(w2a)
    wb2 = jnp.transpose(w2b, (2, 1, 0))
    wc2 = jnp.transpose(w2c, (2, 1, 0))

    ns = 2
    bh = bt // ns
    rh = bh * N
    merged_parts = []
    for s in range(ns):
        gidx3, table1, xyzp = _select(xyz2, pts_flat, s * bh, bh)
        g1 = _sc_gather(gidx3, table1).reshape(rh // 128, 8, 128, 128)
        table2 = _run_chain(_chain1_body, g1, xyzp, wa1, wb1, wc1,
                            b1a.reshape(1, -1), b1b.reshape(1, -1),
                            b1c.reshape(1, -1), None, 0, 128)
        g2 = _sc_gather(gidx3, table2).reshape(rh // 128, 8, 128, 128)
        merged_parts.append(
            _run_chain(_chain2_body, g2, xyzp, wa2, wb2, wc2,
                       b2a.reshape(1, -1), b2b.reshape(1, -1),
                       b2c.reshape(1, -1), pts_flat,
                       s * rh // MBLK, 128))
    merged = jnp.concatenate(merged_parts, axis=0)
    return (xyz, merged.reshape(B, T, N, 128))


# speedup vs baseline: 1.1158x; 1.0118x over previous
import functools

import jax
import jax.numpy as jnp
from jax import lax
from jax.experimental import pallas as pl
from jax.experimental.pallas import tpu as pltpu
from jax.experimental.pallas import tpu_sc as plsc

RADIUS = 0.2
NBLK = 512
MBLK = 1024
_INTERPRET = False


def _select_body(n_total, xyzn_ref, xyzt_ref, pts_ref,
                 gidx_ref, table_ref, xyzp_ref):
    b = pl.program_id(0)
    nb = pl.program_id(1)
    xyzn = xyzn_ref[0]
    xyzt = xyzt_ref[0]
    judge = jnp.float32(RADIUS * RADIUS)
    big = jnp.float32(1e10)
    dx = xyzt[0:1, :] - xyzn[:, 0:1]
    dy = xyzt[1:2, :] - xyzn[:, 1:2]
    dz = xyzt[2:3, :] - xyzn[:, 2:3]
    dist = (dx * dx + dy * dy) + dz * dz
    db = jnp.where((dist > 1e-10) & (dist < judge), dist, big)
    mx, my, mz = dx >= 0, dy >= 0, dz >= 0
    a1 = jnp.where(mx, db, big)
    a0 = jnp.where(mx, big, db)
    b00 = jnp.where(my, big, a0)
    b01 = jnp.where(my, a0, big)
    b10 = jnp.where(my, big, a1)
    b11 = jnp.where(my, a1, big)
    leaves = []
    for bb in (b00, b01, b10, b11):
        leaves.append(jnp.where(mz, big, bb))
        leaves.append(jnp.where(mz, bb, big))
    nlanes = 128
    nch = n_total // nlanes
    lane_iota = lax.broadcasted_iota(jnp.int32, (NBLK, nlanes), 1)
    nglob = nb * NBLK + lax.broadcasted_iota(jnp.int32, (NBLK, 1), 0)
    cols = []
    for lf in leaves:
        best = lf[:, 0:nlanes]
        colarg = jnp.zeros((NBLK, nlanes), jnp.int32)
        for c in range(1, nch):
            v = lf[:, c * nlanes:(c + 1) * nlanes]
            lt = v < best
            best = jnp.where(lt, v, best)
            colarg = jnp.where(lt, jnp.int32(c), colarg)
        mv = jnp.min(best, axis=1, keepdims=True)
        im = jnp.min(jnp.where(best == mv, colarg * nlanes + lane_iota,
                               jnp.int32(n_total)), axis=1, keepdims=True)
        cols.append(jnp.where(mv < judge, im, nglob))
    idx = jnp.concatenate(cols, axis=1)
    idxt = jnp.transpose(idx + b * n_total, (1, 0))
    gidx_ref[...] = jnp.transpose(
        jnp.reshape(idxt, (8, NBLK // 128, 128)), (1, 0, 2))
    zpad13 = jnp.zeros((NBLK, 13), jnp.float32)
    zpad48 = jnp.zeros((NBLK, 48), jnp.float32)
    xyzp = jnp.concatenate([xyzn, zpad13], axis=1)
    xyzp_ref[...] = xyzp
    table_ref[...] = jnp.concatenate([xyzp, pts_ref[...], zpad48], axis=1)


def _select(xyz2, pts_flat, b0, bh):
    n = xyz2.shape[1]
    rows = bh * n
    xyzt = jnp.transpose(xyz2, (0, 2, 1))
    nb_per_b = n // NBLK
    grid = (bh, nb_per_b)
    return pl.pallas_call(
        functools.partial(_select_body, n),
        grid=grid,
        in_specs=[
            pl.BlockSpec((1, NBLK, 3), lambda b, nb: (b0 + b, nb, 0)),
            pl.BlockSpec((1, 3, n), lambda b, nb: (b0 + b, 0, 0)),
            pl.BlockSpec((NBLK, 64),
                         lambda b, nb: ((b0 + b) * nb_per_b + nb, 0)),
        ],
        out_specs=[
            pl.BlockSpec((NBLK // 128, 8, 128),
                         lambda b, nb: (b * nb_per_b + nb, 0, 0)),
            pl.BlockSpec((NBLK, 128), lambda b, nb: (b * nb_per_b + nb, 0)),
            pl.BlockSpec((NBLK, 16), lambda b, nb: (b * nb_per_b + nb, 0)),
        ],
        out_shape=[
            jax.ShapeDtypeStruct((rows // 128, 8, 128), jnp.int32),
            jax.ShapeDtypeStruct((rows, 128), jnp.float32),
            jax.ShapeDtypeStruct((rows, 16), jnp.float32),
        ],
        interpret=_INTERPRET,
    )(xyz2, xyzt, pts_flat)


def _make_sc_gather(nslab, d):
    nw = 32
    nchunk_total = nslab * 8
    nchunk = nchunk_total // nw
    assert nchunk * 128 * d * 4 <= 500_000, "TileSpmem overflow"
    mesh = plsc.VectorSubcoreMesh(core_axis_name="c", subcore_axis_name="s")

    assert nchunk < 8 and 8 % nchunk == 0 or nchunk % 8 == 0
    idx_shape = ((nchunk // 8, 8, 128) if nchunk >= 8 else (nchunk, 128))

    @functools.partial(
        pl.kernel,
        mesh=mesh,
        out_type=jax.ShapeDtypeStruct((nchunk_total * 128, d), jnp.float32),
        scratch_types=[
            pltpu.VMEM(idx_shape, jnp.int32),
            pltpu.VMEM((nchunk * 128, d), jnp.float32),
            pltpu.SemaphoreType.DMA,
        ],
    )
    def gk(gidx_hbm, table_hbm, out_hbm, idx_v, rows_v, sem):
        wid = lax.axis_index("s") * 2 + lax.axis_index("c")
        if nchunk >= 8:
            pltpu.sync_copy(gidx_hbm.at[pl.ds(wid * (nchunk // 8),
                                              nchunk // 8)], idx_v)
        else:
            k0 = (wid * nchunk) // 8
            t0 = (wid * nchunk) % 8
            pltpu.sync_copy(gidx_hbm.at[k0, pl.ds(t0, nchunk)], idx_v)
        cps = [
            pltpu.async_copy(
                table_hbm.at[idx_v.at[c // 8, c % 8] if nchunk >= 8
                             else idx_v.at[c]],
                rows_v.at[pl.ds(c * 128, 128)], sem)
            for c in range(nchunk)
        ]
        for c in cps:
            c.wait()
        pltpu.sync_copy(rows_v,
                        out_hbm.at[pl.ds(wid * nchunk * 128, nchunk * 128)])

    return gk


def _sc_gather(gidx3, table):
    return _make_sc_gather(gidx3.shape[0], table.shape[1])(gidx3, table)


def _chain_core(g_ref, xyzp_ref, wa_ref, wb_ref, wc_ref,
                ba_ref, bb_ref, bc_ref):
    xyzp = xyzp_ref[...]
    c0 = (ba_ref[...]
          - jnp.dot(xyzp, wa_ref[0, 0:16, :],
                    preferred_element_type=jnp.float32)
          - jnp.dot(xyzp, wa_ref[1, 0:16, :],
                    preferred_element_type=jnp.float32))
    nsl = MBLK // 128
    x1 = []
    for w in range(4):
        taps = []
        for k in range(2):
            t = 2 * w + k
            taps.append(jnp.concatenate([g_ref[s, t] for s in range(nsl)],
                                        axis=0))
        s = (jnp.dot(taps[0], wa_ref[0], preferred_element_type=jnp.float32)
             + jnp.dot(taps[1], wa_ref[1],
                       preferred_element_type=jnp.float32))
        x1.append(s + c0)
    x2 = []
    for w in range(2):
        s = (jnp.dot(x1[2 * w], wb_ref[0], preferred_element_type=jnp.float32)
             + jnp.dot(x1[2 * w + 1], wb_ref[1],
                       preferred_element_type=jnp.float32))
        x2.append(s + bb_ref[...])
    return (jnp.dot(x2[0], wc_ref[0], preferred_element_type=jnp.float32)
            + jnp.dot(x2[1], wc_ref[1], preferred_element_type=jnp.float32)
            + bc_ref[...])


def _chain1_body(g_ref, xyzp_ref, wa_ref, wb_ref, wc_ref,
                 ba_ref, bb_ref, bc_ref, out_ref):
    x3 = _chain_core(g_ref, xyzp_ref, wa_ref, wb_ref, wc_ref,
                     ba_ref, bb_ref, bc_ref)
    zpad48 = jnp.zeros((x3.shape[0], 48), jnp.float32)
    out_ref[...] = jnp.concatenate([xyzp_ref[...], x3, zpad48], axis=1)


def _chain2_body(g_ref, xyzp_ref, wa_ref, wb_ref, wc_ref,
                 ba_ref, bb_ref, bc_ref, pts_ref, out_ref):
    x3 = _chain_core(g_ref, xyzp_ref, wa_ref, wb_ref, wc_ref,
                     ba_ref, bb_ref, bc_ref)
    out_ref[...] = jax.nn.relu(jnp.concatenate([x3, pts_ref[...]], axis=1))


def _run_chain(body, g4, xyzp, wa, wb, wc, ba, bb, bc,
               extra, extra_off, out_cols):
    rows = xyzp.shape[0]
    grid = (rows // MBLK,)
    nsl = MBLK // 128
    full = lambda i: (0, 0, 0)
    specs = [
        pl.BlockSpec((nsl, 8, 128, g4.shape[3]), lambda i: (i, 0, 0, 0)),
        pl.BlockSpec((MBLK, 16), lambda i: (i, 0)),
        pl.BlockSpec(wa.shape, full),
        pl.BlockSpec(wb.shape, full),
        pl.BlockSpec(wc.shape, full),
        pl.BlockSpec((1, 64), lambda i: (0, 0)),
        pl.BlockSpec((1, 64), lambda i: (0, 0)),
        pl.BlockSpec((1, 64), lambda i: (0, 0)),
    ]
    args = [g4, xyzp, wa, wb, wc, ba, bb, bc]
    if extra is not None:
        specs.append(pl.BlockSpec((MBLK, 64),
                                  lambda i: (extra_off + i, 0)))
        args.append(extra)
    return pl.pallas_call(
        body,
        grid=grid,
        in_specs=specs,
        out_specs=pl.BlockSpec((MBLK, out_cols), lambda i: (i, 0)),
        out_shape=jax.ShapeDtypeStruct((rows, out_cols), jnp.float32),
        interpret=_INTERPRET,
    )(*args)


def _prep_tap_weights(w):
    o, c, _ = w.shape
    out = jnp.zeros((2, 128, o), jnp.float32)
    wt = jnp.transpose(w, (2, 1, 0))
    out = out.at[:, 0:3, :].set(wt[:, 0:3, :])
    out = out.at[:, 16:16 + (c - 3), :].set(wt[:, 3:, :])
    return out


def kernel(xyz, points, w1a, b1a, w1b, b1b, w1c, b1c,
           w2a, b2a, w2b, b2b, w2c, b2c):
    B, T, N, _ = xyz.shape
    bt = B * T
    rows = bt * N
    xyz2 = xyz.reshape(bt, N, 3)
    pts_flat = points.reshape(rows, -1)

    wa1 = _prep_tap_weights(w1a)
    wb1 = jnp.transpose(w1b, (2, 1, 0))
    wc1 = jnp.transpose(w1c, (2, 1, 0))
    wa2 = _prep_tap_weights(w2a)
    wb2 = jnp.transpose(w2b, (2, 1, 0))
    wc2 = jnp.transpose(w2c, (2, 1, 0))

    ns = 2
    bh = bt // ns
    rh = bh * N
    merged_parts = []
    for s in range(ns):
        gidx3, table1, xyzp = _select(xyz2, pts_flat, s * bh, bh)
        g1 = _sc_gather(gidx3, table1).reshape(rh // 128, 8, 128, 128)
        table2 = _run_chain(_chain1_body, g1, xyzp, wa1, wb1, wc1,
                            b1a.reshape(1, -1), b1b.reshape(1, -1),
                            b1c.reshape(1, -1), None, 0, 128)
        g2 = _sc_gather(gidx3, table2).reshape(rh // 128, 8, 128, 128)
        merged_parts.append(
            _run_chain(_chain2_body, g2, xyzp, wa2, wb2, wc2,
                       b2a.reshape(1, -1), b2b.reshape(1, -1),
                       b2c.reshape(1, -1), pts_flat,
                       s * rh // MBLK, 128))
    merged = jnp.concatenate(merged_parts, axis=0)
    return (xyz, merged.reshape(B, T, N, 128))


# speedup vs baseline: 1.1158x; 1.0001x over previous
import functools

import jax
import jax.numpy as jnp
from jax import lax
from jax.experimental import pallas as pl
from jax.experimental.pallas import tpu as pltpu
from jax.experimental.pallas import tpu_sc as plsc

RADIUS = 0.2
NBLK = 512
MBLK = 1024


def _select_body(n_total, xyzn_ref, xyzt_ref, pts_ref,
                 gidx_ref, table_ref, xyzp_ref):
    b = pl.program_id(0)
    nb = pl.program_id(1)
    xyzn = xyzn_ref[0]
    xyzt = xyzt_ref[0]
    judge = jnp.float32(RADIUS * RADIUS)
    big = jnp.float32(1e10)
    dx = xyzt[0:1, :] - xyzn[:, 0:1]
    dy = xyzt[1:2, :] - xyzn[:, 1:2]
    dz = xyzt[2:3, :] - xyzn[:, 2:3]
    dist = (dx * dx + dy * dy) + dz * dz
    db = jnp.where((dist > 1e-10) & (dist < judge), dist, big)
    mx, my, mz = dx >= 0, dy >= 0, dz >= 0
    a1 = jnp.where(mx, db, big)
    a0 = jnp.where(mx, big, db)
    b00 = jnp.where(my, big, a0)
    b01 = jnp.where(my, a0, big)
    b10 = jnp.where(my, big, a1)
    b11 = jnp.where(my, a1, big)
    leaves = []
    for bb in (b00, b01, b10, b11):
        leaves.append(jnp.where(mz, big, bb))
        leaves.append(jnp.where(mz, bb, big))
    nlanes = 128
    nch = n_total // nlanes
    lane_iota = lax.broadcasted_iota(jnp.int32, (NBLK, nlanes), 1)
    nglob = nb * NBLK + lax.broadcasted_iota(jnp.int32, (NBLK, 1), 0)
    cols = []
    for lf in leaves:
        best = lf[:, 0:nlanes]
        colarg = jnp.zeros((NBLK, nlanes), jnp.int32)
        for c in range(1, nch):
            v = lf[:, c * nlanes:(c + 1) * nlanes]
            lt = v < best
            best = jnp.where(lt, v, best)
            colarg = jnp.where(lt, jnp.int32(c), colarg)
        mv = jnp.min(best, axis=1, keepdims=True)
        im = jnp.min(jnp.where(best == mv, colarg * nlanes + lane_iota,
                               jnp.int32(n_total)), axis=1, keepdims=True)
        cols.append(jnp.where(mv < judge, im, nglob))
    idx = jnp.concatenate(cols, axis=1)
    idxt = jnp.transpose(idx + b * n_total, (1, 0))
    gidx_ref[...] = jnp.transpose(
        jnp.reshape(idxt, (8, NBLK // 128, 128)), (1, 0, 2))
    zpad13 = jnp.zeros((NBLK, 13), jnp.float32)
    zpad48 = jnp.zeros((NBLK, 48), jnp.float32)
    xyzp = jnp.concatenate([xyzn, zpad13], axis=1)
    xyzp_ref[...] = xyzp
    table_ref[...] = jnp.concatenate([xyzp, pts_ref[...], zpad48], axis=1)


def _select(xyz2, pts_flat, b0, bh):
    n = xyz2.shape[1]
    rows = bh * n
    xyzt = jnp.transpose(xyz2, (0, 2, 1))
    nb_per_b = n // NBLK
    grid = (bh, nb_per_b)
    return pl.pallas_call(
        functools.partial(_select_body, n),
        grid=grid,
        in_specs=[
            pl.BlockSpec((1, NBLK, 3), lambda b, nb: (b0 + b, nb, 0)),
            pl.BlockSpec((1, 3, n), lambda b, nb: (b0 + b, 0, 0)),
            pl.BlockSpec((NBLK, 64),
                         lambda b, nb: ((b0 + b) * nb_per_b + nb, 0)),
        ],
        out_specs=[
            pl.BlockSpec((NBLK // 128, 8, 128),
                         lambda b, nb: (b * nb_per_b + nb, 0, 0)),
            pl.BlockSpec((NBLK, 128), lambda b, nb: (b * nb_per_b + nb, 0)),
            pl.BlockSpec((NBLK, 16), lambda b, nb: (b * nb_per_b + nb, 0)),
        ],
        out_shape=[
            jax.ShapeDtypeStruct((rows // 128, 8, 128), jnp.int32),
            jax.ShapeDtypeStruct((rows, 128), jnp.float32),
            jax.ShapeDtypeStruct((rows, 16), jnp.float32),
        ],
    )(xyz2, xyzt, pts_flat)


def _make_sc_gather(nslab, d):
    nw = 32
    nchunk_total = nslab * 8
    nchunk = nchunk_total // nw
    assert nchunk * 128 * d * 4 <= 500_000, "TileSpmem overflow"
    mesh = plsc.VectorSubcoreMesh(core_axis_name="c", subcore_axis_name="s")

    assert nchunk < 8 and 8 % nchunk == 0 or nchunk % 8 == 0
    idx_shape = ((nchunk // 8, 8, 128) if nchunk >= 8 else (nchunk, 128))

    @functools.partial(
        pl.kernel,
        mesh=mesh,
        out_type=jax.ShapeDtypeStruct((nchunk_total * 128, d), jnp.float32),
        scratch_types=[
            pltpu.VMEM(idx_shape, jnp.int32),
            pltpu.VMEM((nchunk * 128, d), jnp.float32),
            pltpu.SemaphoreType.DMA,
        ],
    )
    def gk(gidx_hbm, table_hbm, out_hbm, idx_v, rows_v, sem):
        wid = lax.axis_index("s") * 2 + lax.axis_index("c")
        if nchunk >= 8:
            pltpu.sync_copy(gidx_hbm.at[pl.ds(wid * (nchunk // 8),
                                              nchunk // 8)], idx_v)
        else:
            k0 = (wid * nchunk) // 8
            t0 = (wid * nchunk) % 8
            pltpu.sync_copy(gidx_hbm.at[k0, pl.ds(t0, nchunk)], idx_v)
        cps = [
            pltpu.async_copy(
                table_hbm.at[idx_v.at[c // 8, c % 8] if nchunk >= 8
                             else idx_v.at[c]],
                rows_v.at[pl.ds(c * 128, 128)], sem)
            for c in range(nchunk)
        ]
        for c in cps:
            c.wait()
        pltpu.sync_copy(rows_v,
                        out_hbm.at[pl.ds(wid * nchunk * 128, nchunk * 128)])

    return gk


def _sc_gather(gidx3, table):
    return _make_sc_gather(gidx3.shape[0], table.shape[1])(gidx3, table)


def _chain_core(g_ref, xyzp_ref, wa_ref, wb_ref, wc_ref,
                ba_ref, bb_ref, bc_ref):
    xyzp = xyzp_ref[...]
    c0 = (ba_ref[...]
          - jnp.dot(xyzp, wa_ref[0, 0:16, :],
                    preferred_element_type=jnp.float32)
          - jnp.dot(xyzp, wa_ref[1, 0:16, :],
                    preferred_element_type=jnp.float32))
    nsl = MBLK // 128
    x1 = []
    for w in range(4):
        taps = []
        for k in range(2):
            t = 2 * w + k
            taps.append(jnp.concatenate([g_ref[s, t] for s in range(nsl)],
                                        axis=0))
        s = (jnp.dot(taps[0], wa_ref[0], preferred_element_type=jnp.float32)
             + jnp.dot(taps[1], wa_ref[1],
                       preferred_element_type=jnp.float32))
        x1.append(s + c0)
    x2 = []
    for w in range(2):
        s = (jnp.dot(x1[2 * w], wb_ref[0], preferred_element_type=jnp.float32)
             + jnp.dot(x1[2 * w + 1], wb_ref[1],
                       preferred_element_type=jnp.float32))
        x2.append(s + bb_ref[...])
    return (jnp.dot(x2[0], wc_ref[0], preferred_element_type=jnp.float32)
            + jnp.dot(x2[1], wc_ref[1], preferred_element_type=jnp.float32)
            + bc_ref[...])


def _chain1_body(g_ref, xyzp_ref, wa_ref, wb_ref, wc_ref,
                 ba_ref, bb_ref, bc_ref, out_ref):
    x3 = _chain_core(g_ref, xyzp_ref, wa_ref, wb_ref, wc_ref,
                     ba_ref, bb_ref, bc_ref)
    zpad48 = jnp.zeros((x3.shape[0], 48), jnp.float32)
    out_ref[...] = jnp.concatenate([xyzp_ref[...], x3, zpad48], axis=1)


def _chain2_body(g_ref, xyzp_ref, wa_ref, wb_ref, wc_ref,
                 ba_ref, bb_ref, bc_ref, pts_ref, out_ref):
    x3 = _chain_core(g_ref, xyzp_ref, wa_ref, wb_ref, wc_ref,
                     ba_ref, bb_ref, bc_ref)
    out_ref[...] = jax.nn.relu(jnp.concatenate([x3, pts_ref[...]], axis=1))


def _run_chain(body, g4, xyzp, wa, wb, wc, ba, bb, bc,
               extra, extra_off, out_cols):
    rows = xyzp.shape[0]
    grid = (rows // MBLK,)
    nsl = MBLK // 128
    full = lambda i: (0, 0, 0)
    specs = [
        pl.BlockSpec((nsl, 8, 128, g4.shape[3]), lambda i: (i, 0, 0, 0)),
        pl.BlockSpec((MBLK, 16), lambda i: (i, 0)),
        pl.BlockSpec(wa.shape, full),
        pl.BlockSpec(wb.shape, full),
        pl.BlockSpec(wc.shape, full),
        pl.BlockSpec((1, 64), lambda i: (0, 0)),
        pl.BlockSpec((1, 64), lambda i: (0, 0)),
        pl.BlockSpec((1, 64), lambda i: (0, 0)),
    ]
    args = [g4, xyzp, wa, wb, wc, ba, bb, bc]
    if extra is not None:
        specs.append(pl.BlockSpec((MBLK, 64),
                                  lambda i: (extra_off + i, 0)))
        args.append(extra)
    return pl.pallas_call(
        body,
        grid=grid,
        in_specs=specs,
        out_specs=pl.BlockSpec((MBLK, out_cols), lambda i: (i, 0)),
        out_shape=jax.ShapeDtypeStruct((rows, out_cols), jnp.float32),
    )(*args)


def _prep_tap_weights(w):
    o, c, _ = w.shape
    out = jnp.zeros((2, 128, o), jnp.float32)
    wt = jnp.transpose(w, (2, 1, 0))
    out = out.at[:, 0:3, :].set(wt[:, 0:3, :])
    out = out.at[:, 16:16 + (c - 3), :].set(wt[:, 3:, :])
    return out


def kernel(xyz, points, w1a, b1a, w1b, b1b, w1c, b1c,
           w2a, b2a, w2b, b2b, w2c, b2c):
    B, T, N, _ = xyz.shape
    bt = B * T
    rows = bt * N
    xyz2 = xyz.reshape(bt, N, 3)
    pts_flat = points.reshape(rows, -1)

    wa1 = _prep_tap_weights(w1a)
    wb1 = jnp.transpose(w1b, (2, 1, 0))
    wc1 = jnp.transpose(w1c, (2, 1, 0))
    wa2 = _prep_tap_weights(w2a)
    wb2 = jnp.transpose(w2b, (2, 1, 0))
    wc2 = jnp.transpose(w2c, (2, 1, 0))

    ns = 2
    bh = bt // ns
    rh = bh * N
    merged_parts = []
    for s in range(ns):
        gidx3, table1, xyzp = _select(xyz2, pts_flat, s * bh, bh)
        g1 = _sc_gather(gidx3, table1).reshape(rh // 128, 8, 128, 128)
        table2 = _run_chain(_chain1_body, g1, xyzp, wa1, wb1, wc1,
                            b1a.reshape(1, -1), b1b.reshape(1, -1),
                            b1c.reshape(1, -1), None, 0, 128)
        g2 = _sc_gather(gidx3, table2).reshape(rh // 128, 8, 128, 128)
        merged_parts.append(
            _run_chain(_chain2_body, g2, xyzp, wa2, wb2, wc2,
                       b2a.reshape(1, -1), b2b.reshape(1, -1),
                       b2c.reshape(1, -1), pts_flat,
                       s * rh // MBLK, 128))
    merged = jnp.concatenate(merged_parts, axis=0)
    return (xyz, merged.reshape(B, T, N, 128))
